# baseline (speedup 1.0000x reference)
import functools

import jax
import jax.numpy as jnp
from jax import lax
from jax.experimental import pallas as pl
from jax.experimental.pallas import tpu as pltpu
from jax.experimental.pallas import tpu_sc as plsc

EPS = 1e-12
LANES = 16
GT = 8

_GATHER_DNUMS = lax.GatherDimensionNumbers(
    offset_dims=(), collapsed_slice_dims=(0,), start_index_map=(0,))


def _lane_rotate(x, k):
    idx = jnp.bitwise_and(lax.iota(jnp.int32, LANES) + k, LANES - 1)
    return lax.gather(x, idx[:, None], _GATHER_DNUMS, slice_sizes=(1,),
                      mode=lax.GatherScatterMode.PROMISE_IN_BOUNDS)


def _lane_allsum(x):
    for k in (8, 4, 2, 1):
        x = x + _lane_rotate(x, k)
    return x


def _rsqrt_vec(x):
    i = lax.bitcast_convert_type(x, jnp.int32)
    i = jnp.int32(0x5F3759DF) - lax.shift_right_logical(i, 1)
    y = lax.bitcast_convert_type(i, jnp.float32)
    for _ in range(3):
        y = y * (1.5 - 0.5 * x * y * y)
    return y


@functools.lru_cache(maxsize=None)
def _build(T, S, D, C):
    info = plsc.get_sparse_core_info()
    NC, NS = info.num_cores, info.num_subcores
    NW = NC * NS
    per_w = T // NW
    n_chunks = per_w // C
    n_pairs = n_chunks // 2
    NV = D // LANES

    mesh = plsc.VectorSubcoreMesh(core_axis_name="c", subcore_axis_name="s")

    @functools.partial(
        pl.kernel,
        mesh=mesh,
        compiler_params=pltpu.CompilerParams(use_tc_tiling_on_sc=False),
        out_type=jax.ShapeDtypeStruct((T, NV, LANES), jnp.float32),
        scratch_types=[
            pltpu.VMEM((per_w,), jnp.int32),
            pltpu.VMEM((C, NV, LANES), jnp.float32),
            pltpu.VMEM((C, NV, LANES), jnp.float32),
            pltpu.VMEM((C, NV, LANES), jnp.float32),
            pltpu.VMEM((C, NV, LANES), jnp.float32),
            pltpu.VMEM((NV, LANES), jnp.float32),
            pltpu.VMEM((NV, LANES), jnp.float32),
            pltpu.SemaphoreType.DMA,
            pltpu.SemaphoreType.DMA,
        ],
    )
    def embed_ln(ids_hbm, wt_hbm, pt_hbm, g_hbm, b_hbm, out_hbm,
                 idx_all, xb0, xb1, pb0, pb1, g_v, b_v, sem0, sem1):
        wid = lax.axis_index("s") * NC + lax.axis_index("c")
        base = wid * per_w
        pltpu.sync_copy(g_hbm, g_v)
        pltpu.sync_copy(b_hbm, b_v)
        pltpu.sync_copy(ids_hbm.at[pl.ds(base, per_w)], idx_all)

        bufs = ((xb0, pb0, sem0), (xb1, pb1, sem1))

        def issue(c, bi):
            xb, pb, sem = bufs[bi]
            fb = base + c * C
            s0 = lax.rem(fb, S)
            pltpu.async_copy(wt_hbm.at[idx_all.at[pl.ds(c * C, C)]], xb, sem)
            pltpu.async_copy(pt_hbm.at[pl.ds(s0, C)], pb, sem)

        def wait(bi):
            xb, pb, sem = bufs[bi]
            pltpu.make_async_copy(
                wt_hbm.at[idx_all.at[pl.ds(0, C)]], xb, sem).wait()
            pltpu.make_async_copy(pt_hbm.at[pl.ds(0, C)], pb, sem).wait()

        def compute(c, bi):
            xb, pb, sem = bufs[bi]
            zero = jnp.zeros((LANES,), jnp.float32)
            for g in range(C // GT):
                t0 = g * GT

                def p_add(j, carry):
                    out = []
                    for t in range(GT):
                        a, q = carry[2 * t], carry[2 * t + 1]
                        x = xb[t0 + t, j] + pb[t0 + t, j]
                        xb[t0 + t, j] = x
                        out.append(a + x)
                        out.append(q + x * x)
                    return tuple(out)

                accs = lax.fori_loop(0, NV, p_add, (zero,) * (2 * GT))

                stats = []
                for t in range(GT):
                    mean = _lane_allsum(accs[2 * t]) * (1.0 / D)
                    var = _lane_allsum(accs[2 * t + 1]) * (1.0 / D) \
                        - mean * mean
                    stats.append(mean)
                    stats.append(_rsqrt_vec(var + EPS))

                def p_norm(j, carry):
                    gj = g_v[j]
                    bj = b_v[j]
                    for t in range(GT):
                        x = xb[t0 + t, j]
                        y = ((x - carry[2 * t]) * carry[2 * t + 1]) * gj + bj
                        xb[t0 + t, j] = y
                    return carry

                lax.fori_loop(0, NV, p_norm, tuple(stats))
            fb = base + c * C
            pltpu.sync_copy(xb, out_hbm.at[pl.ds(fb, C)])

        def pair(p, issue_next):
            c0 = 2 * p
            issue(c0 + 1, 1)
            wait(0)
            compute(c0, 0)
            if issue_next:
                issue(c0 + 2, 0)
            wait(1)
            compute(c0 + 1, 1)

        issue(0, 0)

        def pair_body(p, _):
            pair(p, True)
            return 0

        lax.fori_loop(0, n_pairs - 1, pair_body, 0)
        pair(n_pairs - 1, False)

    return embed_ln


def kernel(input_ids, word_table, pos_table, gamma, beta):
    B, S = input_ids.shape
    V, D = word_table.shape
    T = B * S
    NV = D // LANES
    ids_flat = input_ids.reshape(T).astype(jnp.int32)
    fn = _build(T, S, D, 32)
    out = fn(ids_flat,
             word_table.reshape(V, NV, LANES),
             pos_table.reshape(pos_table.shape[0], NV, LANES),
             gamma.reshape(NV, LANES),
             beta.reshape(NV, LANES))
    return out.reshape(B, S, D)


# speedup vs baseline: 17.8547x; 17.8547x over previous
import functools

import jax
import jax.numpy as jnp
from jax import lax
from jax.experimental import pallas as pl
from jax.experimental.pallas import tpu as pltpu
from jax.experimental.pallas import tpu_sc as plsc

EPS = 1e-12
LANES = 16
GT = 8

_GATHER_DNUMS = lax.GatherDimensionNumbers(
    offset_dims=(), collapsed_slice_dims=(0,), start_index_map=(0,))


def _lane_rotate(x, k):
    idx = jnp.bitwise_and(lax.iota(jnp.int32, LANES) + k, LANES - 1)
    return lax.gather(x, idx[:, None], _GATHER_DNUMS, slice_sizes=(1,),
                      mode=lax.GatherScatterMode.PROMISE_IN_BOUNDS)


def _lane_allsum(x):
    for k in (8, 4, 2, 1):
        x = x + _lane_rotate(x, k)
    return x


def _rsqrt_vec(x):
    i = lax.bitcast_convert_type(x, jnp.int32)
    i = jnp.int32(0x5F3759DF) - lax.shift_right_logical(i, 1)
    y = lax.bitcast_convert_type(i, jnp.float32)
    for _ in range(3):
        y = y * (1.5 - 0.5 * x * y * y)
    return y


@functools.lru_cache(maxsize=None)
def _build(B, S, D, C):
    info = plsc.get_sparse_core_info()
    NC, NS = info.num_cores, info.num_subcores
    NW = NC * NS
    T = B * S
    per_w = T // NW
    n_chunks = per_w // C
    n_pairs = n_chunks // 2
    NV = D // LANES

    mesh = plsc.VectorSubcoreMesh(core_axis_name="c", subcore_axis_name="s")

    @functools.partial(
        pl.kernel,
        mesh=mesh,
        out_type=jax.ShapeDtypeStruct((B, S, D), jnp.float32),
        scratch_types=[
            pltpu.VMEM((per_w,), jnp.int32),
            pltpu.VMEM((C, D), jnp.float32),
            pltpu.VMEM((C, D), jnp.float32),
            pltpu.VMEM((C, D), jnp.float32),
            pltpu.VMEM((C, D), jnp.float32),
            pltpu.VMEM((D,), jnp.float32),
            pltpu.VMEM((D,), jnp.float32),
            pltpu.SemaphoreType.DMA,
            pltpu.SemaphoreType.DMA,
        ],
    )
    def embed_ln(ids_hbm, wt_hbm, pt_hbm, g_hbm, b_hbm, out_hbm,
                 idx_all, xb0, xb1, pb0, pb1, g_v, b_v, sem0, sem1):
        wid = lax.axis_index("s") * NC + lax.axis_index("c")
        base = wid * per_w
        b_idx = lax.div(base, S)
        col0 = lax.rem(base, S)
        pltpu.sync_copy(g_hbm, g_v)
        pltpu.sync_copy(b_hbm, b_v)
        pltpu.sync_copy(ids_hbm.at[b_idx, pl.ds(col0, per_w)], idx_all)

        bufs = ((xb0, pb0, sem0), (xb1, pb1, sem1))

        def issue(c, bi):
            xb, pb, sem = bufs[bi]
            s0 = col0 + c * C
            pltpu.async_copy(wt_hbm.at[idx_all.at[pl.ds(c * C, C)]], xb, sem)
            pltpu.async_copy(pt_hbm.at[pl.ds(s0, C)], pb, sem)

        def wait(bi):
            xb, pb, sem = bufs[bi]
            pltpu.make_async_copy(
                wt_hbm.at[idx_all.at[pl.ds(0, C)]], xb, sem).wait()
            pltpu.make_async_copy(pt_hbm.at[pl.ds(0, C)], pb, sem).wait()

        def compute(c, bi):
            xb, pb, sem = bufs[bi]
            zero = jnp.zeros((LANES,), jnp.float32)
            for g in range(C // GT):
                t0 = g * GT

                def p_add(j, carry):
                    sl = pl.ds(j * LANES, LANES)
                    out = []
                    for t in range(GT):
                        a, q = carry[2 * t], carry[2 * t + 1]
                        x = xb[t0 + t, sl] + pb[t0 + t, sl]
                        xb[t0 + t, sl] = x
                        out.append(a + x)
                        out.append(q + x * x)
                    return tuple(out)

                accs = lax.fori_loop(0, NV, p_add, (zero,) * (2 * GT))

                stats = []
                for t in range(GT):
                    mean = _lane_allsum(accs[2 * t]) * (1.0 / D)
                    var = _lane_allsum(accs[2 * t + 1]) * (1.0 / D) \
                        - mean * mean
                    stats.append(mean)
                    stats.append(_rsqrt_vec(var + EPS))

                def p_norm(j, carry):
                    sl = pl.ds(j * LANES, LANES)
                    gj = g_v[sl]
                    bj = b_v[sl]
                    for t in range(GT):
                        x = xb[t0 + t, sl]
                        y = ((x - carry[2 * t]) * carry[2 * t + 1]) * gj + bj
                        xb[t0 + t, sl] = y
                    return carry

                lax.fori_loop(0, NV, p_norm, tuple(stats))
            s0 = col0 + c * C
            pltpu.sync_copy(xb, out_hbm.at[b_idx, pl.ds(s0, C)])

        def pair(p, issue_next):
            c0 = 2 * p
            issue(c0 + 1, 1)
            wait(0)
            compute(c0, 0)
            if issue_next:
                issue(c0 + 2, 0)
            wait(1)
            compute(c0 + 1, 1)

        issue(0, 0)

        def pair_body(p, _):
            pair(p, True)
            return 0

        lax.fori_loop(0, n_pairs - 1, pair_body, 0)
        pair(n_pairs - 1, False)

    return embed_ln


def kernel(input_ids, word_table, pos_table, gamma, beta):
    B, S = input_ids.shape
    V, D = word_table.shape
    fn = _build(B, S, D, 32)
    return fn(input_ids.astype(jnp.int32), word_table, pos_table, gamma, beta)
